# baseline (speedup 1.0000x reference)
import functools

import jax
import jax.numpy as jnp
import numpy as np
from jax import lax
from jax.experimental import pallas as pl
from jax.experimental.pallas import tpu as pltpu
from jax.experimental.pallas import tpu_sc as plsc

BN = 256
BE = 256
CH = 96


def _proj_body(T, nt_ref, x_ref, kw_ref, kb_ref, qw_ref, qb_ref, vw_ref, vb_ref,
               wc_ref, nta_ref, nta1_ref, kvc_ref, qc_ref):
    xb = x_ref[...]
    ntb = nt_ref[...]
    outs = []
    for (w_ref, b_ref) in ((kw_ref, kb_ref), (qw_ref, qb_ref), (vw_ref, vb_ref)):
        acc = jnp.zeros((xb.shape[0], w_ref.shape[2]), jnp.float32)
        for t in range(T):
            m = jnp.dot(xb, w_ref[t], preferred_element_type=jnp.float32)
            m = m + b_ref[t][None, :]
            acc = jnp.where(ntb == t, m, acc)
        outs.append(acc)
    kacc, qacc, vacc = outs
    wc = wc_ref[...]
    c0 = jnp.dot(kacc, wc[:, 0:8], preferred_element_type=jnp.float32)
    c1 = jnp.dot(qacc, wc[:, 8:16], preferred_element_type=jnp.float32)
    nth = nt_ref[..., 0:8]
    s0 = jnp.zeros_like(c0)
    s1 = jnp.zeros_like(c1)
    for t in range(T):
        s0 = jnp.where(nth == t, nta_ref[t], s0)
        s1 = jnp.where(nth == t, nta1_ref[t], s1)
    c01p = jnp.concatenate(
        [c0 * s0, c1 * s1, jnp.zeros((xb.shape[0], 112), jnp.float32)], axis=1)
    kvc_ref[..., 0:128] = kacc
    kvc_ref[..., 128:256] = vacc
    kvc_ref[..., 256:384] = c01p
    pid = pl.program_id(0)
    row = lax.broadcasted_iota(jnp.int32, c01p.shape, 0).astype(jnp.float32)
    lane = lax.broadcasted_iota(jnp.int32, c01p.shape, 1)
    basef = lax.convert_element_type(pid * c01p.shape[0], jnp.float32)
    ids = jnp.where(lane == 16, row + basef, 0.0)
    qc_ref[..., 0:128] = qacc
    qc_ref[..., 128:256] = c01p + ids


def _edge_body(R, rel_ref, kvc_ref, qc_ref, a_ref, m_ref,
               sel_ref, beta_ref, lg_ref, le_ref, mv_ref, seg_ref, dst_ref):
    i = pl.program_id(0)
    r = rel_ref[i]
    kg = kvc_ref[..., 0:128]
    vg = kvc_ref[..., 128:256]
    qg = qc_ref[..., 0:128]
    z = jnp.dot(kg, a_ref[0], preferred_element_type=jnp.float32)
    att = jnp.dot(z * qg, sel_ref[...], preferred_element_type=jnp.float32)
    att2 = kvc_ref[..., 256:264] + qc_ref[..., 136:144]
    att2 = jnp.where(att2 >= 0, att2, 0.01 * att2)
    lg = att + beta_ref[0] * att2
    lg_ref[...] = jnp.transpose(lg)
    le_ref[...] = jnp.concatenate(
        [lg, jnp.zeros((lg.shape[0], 8), jnp.float32)], axis=1)
    mv_ref[...] = jnp.dot(vg, m_ref[0], preferred_element_type=jnp.float32)
    dt = jnp.transpose(qc_ref[..., 144:145]).astype(jnp.int32)
    dst_ref[0] = dt
    seg_ref[0] = dt * R + r


def _final_body(T, nt_ref, t_ref, d0_ref, x_ref, aw_ref, ab_ref, skip_ref, o_ref):
    pres = (d0_ref[...] > 0).astype(jnp.float32)
    dn = jnp.maximum(jnp.sum(pres, axis=1, keepdims=True), 1.0)
    tb = (t_ref[0] + t_ref[1]) / dn
    ntb = nt_ref[...]
    acc = jnp.zeros_like(tb)
    al = jnp.zeros_like(tb)
    for t in range(T):
        m = jnp.dot(tb, aw_ref[t], preferred_element_type=jnp.float32) + ab_ref[t][None, :]
        acc = jnp.where(ntb == t, m, acc)
        al = jnp.where(ntb == t, jax.nn.sigmoid(skip_ref[t]), al)
    o_ref[...] = acc * al + x_ref[...] * (1.0 - al)


def _cnt_body(nt_ref, o_ref, acc_ref):
    i = pl.program_id(0)

    @pl.when(i == 0)
    def _():
        acc_ref[...] = jnp.zeros_like(acc_ref)

    etb = jnp.broadcast_to(nt_ref[0].astype(jnp.float32), (40, 512))
    rid = lax.broadcasted_iota(jnp.int32, (40, 512), 0).astype(jnp.float32)
    oh = (etb == rid).astype(jnp.float32)
    acc_ref[...] = acc_ref[...] + jnp.broadcast_to(
        jnp.sum(oh, axis=1, keepdims=True), (40, 128))
    o_ref[...] = acc_ref[...]


def _rank_body(et_ref, po_ref, lt_ref, pp_ref, acc_ref):
    i = pl.program_id(0)

    @pl.when(i == 0)
    def _():
        acc_ref[...] = jnp.zeros_like(acc_ref)

    etb = jnp.broadcast_to(et_ref[0].astype(jnp.float32), (40, 512))
    rid = lax.broadcasted_iota(jnp.int32, (40, 512), 0).astype(jnp.float32)
    oh = (etb == rid).astype(jnp.float32)
    cum = jnp.dot(oh.astype(jnp.bfloat16), lt_ref[...],
                  preferred_element_type=jnp.float32)
    rank = jnp.sum(oh * cum, axis=0, keepdims=True)
    base = jnp.sum(oh * acc_ref[...], axis=0, keepdims=True)
    posel = jnp.sum(oh * po_ref[...], axis=0, keepdims=True)
    pp_ref[0] = (rank + base + posel).astype(jnp.int32)
    acc_ref[...] = acc_ref[...] + jnp.broadcast_to(
        jnp.sum(oh, axis=1, keepdims=True), (40, 512))


def _make_gather(P, OUT, CW):
    NW = 32
    PT = P // NW
    NIT = PT // CH
    mesh = plsc.VectorSubcoreMesh(core_axis_name="c", subcore_axis_name="s")
    f32 = jnp.float32

    @functools.partial(
        pl.kernel, mesh=mesh,
        out_type=[
            jax.ShapeDtypeStruct((P, 3 * OUT), f32),
            jax.ShapeDtypeStruct((P, 2 * OUT), f32),
        ],
        scratch_types=[
            pltpu.VMEM((CH,), jnp.int32),
            pltpu.VMEM((CH,), jnp.int32),
            pltpu.VMEM((CH,), jnp.int32),
            pltpu.VMEM((CH,), jnp.int32),
            pltpu.VMEM((CH,), jnp.int32),
            pltpu.VMEM((CH,), jnp.int32),
            pltpu.VMEM((CH, 3 * OUT), f32),
            pltpu.VMEM((CH, 3 * OUT), f32),
            pltpu.VMEM((CH, 2 * OUT), f32),
            pltpu.VMEM((CH, 2 * OUT), f32),
            pltpu.SemaphoreType.DMA,
            pltpu.SemaphoreType.DMA,
            pltpu.SemaphoreType.DMA,
            pltpu.SemaphoreType.DMA,
        ],
    )
    def g(ssrc_h, sdst_h, pos_h, kvct_h, qct_h, kvc_h, qc_h,
          isrc, isrc2, idst, idst2, ipos, ipos2, rbuf, rbuf2, qbuf, qbuf2,
          sem, sem2, sem3, sem4):
        c = lax.axis_index("c")
        s = lax.axis_index("s")
        base = (s * 2 + c) * PT

        def one(off, si, di, pi, rb, qb, sg1, sg2):
            pltpu.sync_copy(ssrc_h.at[pl.ds(off, CH)], si)
            pltpu.sync_copy(sdst_h.at[pl.ds(off, CH)], di)
            pltpu.sync_copy(pos_h.at[pl.ds(off, CH)], pi)
            gk = pltpu.async_copy(kvct_h.at[si], rb, sg1)
            gq = pltpu.async_copy(qct_h.at[di], qb, sg2)
            return gk, gq

        def scat(pi, rb, qb, sg1, sg2):
            sk = pltpu.async_copy(rb, kvc_h.at[pi], sg1)
            sq = pltpu.async_copy(qb, qc_h.at[pi], sg2)
            return sk, sq

        def pair(p, carry):
            offa = base + (2 * p) * CH
            offb = offa + CH
            ga = one(offa, isrc, idst, ipos, rbuf, qbuf, sem, sem2)
            gb = one(offb, isrc2, idst2, ipos2, rbuf2, qbuf2, sem3, sem4)
            ga[0].wait()
            ga[1].wait()
            sa = scat(ipos, rbuf, qbuf, sem, sem2)
            gb[0].wait()
            gb[1].wait()
            sb = scat(ipos2, rbuf2, qbuf2, sem3, sem4)
            sa[0].wait()
            sa[1].wait()
            sb[0].wait()
            sb[1].wait()
            return carry

        lax.fori_loop(0, NIT // 2, pair, 0)
        if NIT % 2:
            off = base + (NIT - 1) * CH
            gk, gq = one(off, isrc, idst, ipos, rbuf, qbuf, sem, sem2)
            gk.wait()
            gq.wait()
            sk, sq = scat(ipos, rbuf, qbuf, sem, sem2)
            sk.wait()
            sq.wait()

    return g


def _make_den(P, Np, R, SEGR, H):
    CH2 = 768
    NIT = P // CH2
    QR = SEGR // 4
    TBL = QR + 16
    mesh = plsc.VectorSubcoreMesh(core_axis_name="c", subcore_axis_name="s")
    f32 = jnp.float32

    @functools.partial(
        pl.kernel, mesh=mesh,
        compiler_params=pltpu.CompilerParams(needs_layout_passes=False),
        out_type=jax.ShapeDtypeStruct((H, SEGR), f32),
        scratch_types=[
            pltpu.VMEM((TBL,), f32),
            pltpu.VMEM((CH2,), jnp.int32),
            pltpu.VMEM((CH2,), f32),
            pltpu.SemaphoreType.DMA,
        ],
    )
    def g(lg_h, seg_h, den_h, tbl, segb, lgb, sem):
        c = lax.axis_index("c")
        s = lax.axis_index("s")
        combo = c * 16 + s
        q = combo // H
        h = combo % H
        qbase = q * QR

        def zrow(i, carry):
            tbl[pl.ds(i * 16, 16)] = jnp.zeros((16,), f32)
            return carry
        lax.fori_loop(0, TBL // 16, zrow, 0)

        def eit(j, carry):
            off = j * CH2
            pltpu.sync_copy(seg_h.at[pl.ds(off, CH2)], segb)
            pltpu.sync_copy(lg_h.at[h, pl.ds(off, CH2)], lgb)
            for v in range(CH2 // 16):
                sv = segb[pl.ds(v * 16, 16)]
                lv = sv - qbase
                ok = (lv >= 0) & (lv < QR)
                li = jnp.where(ok, lv, QR)
                ex = jnp.exp(lgb[pl.ds(v * 16, 16)])
                plsc.addupdate_scatter(tbl, [li], ex)
            return carry
        lax.fori_loop(0, NIT, eit, 0)

        pltpu.sync_copy(tbl.at[pl.ds(0, QR)], den_h.at[h, pl.ds(qbase, QR)])

    return g


def _make_scatter(P, Np, OUT, H):
    P2 = P // 2
    PT = P2 // 16
    NIT = PT // CH
    TR = Np // 16
    mesh = plsc.VectorSubcoreMesh(core_axis_name="c", subcore_axis_name="s")
    f32 = jnp.float32

    @functools.partial(
        pl.kernel, mesh=mesh,
        out_type=jax.ShapeDtypeStruct((2, Np, OUT), f32),
        compiler_params=pltpu.CompilerParams(needs_layout_passes=False),
        scratch_types=[
            pltpu.VMEM_SHARED((Np, OUT), f32),
            pltpu.VMEM((32, OUT), f32),
            pltpu.VMEM((CH,), jnp.int32),
            pltpu.VMEM((CH, 16), f32),
            pltpu.VMEM((CH, 16), f32),
            pltpu.VMEM((CH, OUT), f32),
            pltpu.SemaphoreType.DMA,
        ],
    )
    def g(lg_h, dst_h, mv_h, denp_h, tp_h, table, zbuf, dstb, lgb,
          denb, mvb, sem):
        c = lax.axis_index("c")
        s = lax.axis_index("s")

        def zrow(i, carry):
            for j in range(OUT // 16):
                zbuf[i, pl.ds(j * 16, 16)] = jnp.zeros((16,), f32)
            return carry
        lax.fori_loop(0, 32, zrow, 0)

        def zit(j, carry):
            pltpu.sync_copy(zbuf, table.at[pl.ds(s * TR + j * 32, 32)])
            return carry
        lax.fori_loop(0, TR // 32, zit, 0)
        plsc.subcore_barrier()

        ebase = c * P2 + s * PT

        def eit(j, carry):
            off = ebase + j * CH
            pltpu.sync_copy(dst_h.at[pl.ds(off, CH)], dstb)
            pltpu.sync_copy(lg_h.at[pl.ds(off, CH)], lgb)
            pltpu.sync_copy(denp_h.at[pl.ds(off, CH)], denb)
            pltpu.sync_copy(mv_h.at[pl.ds(off, CH)], mvb)

            def erow(i, carry2):
                a = jnp.exp(lgb[i]) / jnp.maximum(denb[i], 1e-9)
                for h in range(H):
                    mvb[i, pl.ds(h * 16, 16)] = mvb[i, pl.ds(h * 16, 16)] * a[h]
                return carry2
            lax.fori_loop(0, CH, erow, 0, unroll=2)
            pltpu.sync_copy(mvb, table.at[dstb], add=True)
            return carry
        lax.fori_loop(0, NIT, eit, 0)
        plsc.subcore_barrier()

        pltpu.sync_copy(table.at[pl.ds(s * TR, TR)],
                        tp_h.at[c, pl.ds(s * TR, TR)])

    return g


def kernel(x, edge_index, edge_type, node_type, k_w, k_b, q_w, q_b, v_w, v_b,
           a_w, a_b, relation_pri, relation_att, relation_msg, node_type_att,
           node_type_att1, skip, weight, attn_fc_w):
    N, IN = x.shape
    T, _, OUT = k_w.shape
    R, H, DK, _ = relation_att.shape
    E = edge_index.shape[1]
    Np = ((N + BN - 1) // BN) * BN
    NBn = Np // BN
    P = E + R * BE
    NB = P // BE

    f32 = jnp.float32
    xp = jnp.pad(x, ((0, Np - N), (0, 0)))
    ntp = jnp.pad(node_type, (0, Np - N)).astype(jnp.int32)
    ntb = jnp.broadcast_to(ntp.astype(f32)[:, None], (Np, OUT))

    w1 = attn_fc_w[:DK]
    w2 = attn_fc_w[DK:]
    eyeh = np.zeros((OUT, 2 * H), np.float32)
    for h in range(H):
        eyeh[h * DK:(h + 1) * DK, h] = 1.0
        eyeh[h * DK:(h + 1) * DK, H + h] = 1.0
    eyeh = jnp.asarray(eyeh)
    wc = jnp.concatenate([eyeh[:, :H] * jnp.tile(w2, H)[:, None],
                          eyeh[:, H:] * jnp.tile(w1, H)[:, None]], axis=1)

    bspec = lambda bs, im: pl.BlockSpec(bs, im)
    full = lambda arr: pl.BlockSpec(arr.shape, lambda *a: tuple(0 for _ in arr.shape))
    grid1 = (NBn,)
    kvc_t, qc_t = pl.pallas_call(
        functools.partial(_proj_body, T),
        grid=grid1,
        in_specs=[
            bspec((BN, OUT), lambda i: (i, 0)),
            bspec((BN, IN), lambda i: (i, 0)),
            full(k_w), full(k_b), full(q_w), full(q_b), full(v_w), full(v_b),
            full(wc),
            pl.BlockSpec(memory_space=pltpu.SMEM),
            pl.BlockSpec(memory_space=pltpu.SMEM),
        ],
        out_specs=[
            bspec((BN, 3 * OUT), lambda i: (i, 0)),
            bspec((BN, 2 * OUT), lambda i: (i, 0)),
        ],
        out_shape=[
            jax.ShapeDtypeStruct((Np, 3 * OUT), f32),
            jax.ShapeDtypeStruct((Np, 2 * OUT), f32),
        ],
    )(ntb, xp, k_w, k_b, q_w, q_b, v_w, v_b, wc, node_type_att, node_type_att1)

    src = edge_index[0]
    dst = edge_index[1]
    EBK = E // 512
    et3 = edge_type.astype(jnp.int32).reshape(EBK, 1, 512)
    cnt40 = pl.pallas_call(
        _cnt_body,
        grid=(EBK,),
        in_specs=[bspec((1, 1, 512), lambda i: (i, 0, 0))],
        out_specs=bspec((40, 128), lambda i: (0, 0)),
        out_shape=jax.ShapeDtypeStruct((40, 128), f32),
        scratch_shapes=[pltpu.VMEM((40, 128), f32)],
    )(et3)
    cnt = cnt40[:R, 0].astype(jnp.int32)
    cap = ((cnt + BE - 1) // BE) * BE
    scap = jnp.cumsum(cap)
    po = jnp.concatenate([jnp.zeros((1,), jnp.int32), scap])[:R]
    total_used = scap[R - 1]
    po40 = jnp.broadcast_to(
        jnp.pad(po.astype(f32), (0, 40 - R))[:, None], (40, 512))
    lt = jnp.asarray(np.triu(np.ones((512, 512), np.float32), 1).T,
                     dtype=jnp.bfloat16)
    ppos3 = pl.pallas_call(
        _rank_body,
        grid=(EBK,),
        in_specs=[
            bspec((1, 1, 512), lambda i: (i, 0, 0)),
            bspec((40, 512), lambda i: (0, 0)),
            bspec((512, 512), lambda i: (0, 0)),
        ],
        out_specs=bspec((1, 1, 512), lambda i: (i, 0, 0)),
        out_shape=jax.ShapeDtypeStruct((EBK, 1, 512), jnp.int32),
        scratch_shapes=[pltpu.VMEM((40, 512), f32)],
    )(et3, po40, lt)
    ppos = ppos3.reshape(E)
    jpad = jnp.arange(R * BE, dtype=jnp.int32)
    rr = jpad // BE
    jj = jpad % BE
    navail = cap - cnt
    validp = jj < navail[rr]
    extra_rank = jnp.cumsum((~validp).astype(jnp.int32)) - 1
    pad_pos = jnp.where(validp, po[rr] + cnt[rr] + jj, total_used + extra_rank)
    ssrc = jnp.concatenate([src, jnp.zeros((R * BE,), jnp.int32)])
    sdst_t = jnp.concatenate([dst, jnp.full((R * BE,), N, jnp.int32)])
    pos_t = jnp.concatenate([ppos, pad_pos.astype(jnp.int32)])
    rel_blocks = jnp.clip(
        jnp.searchsorted(scap, jnp.arange(NB, dtype=jnp.int32) * BE,
                         side="right"), 0, R - 1).astype(jnp.int32)

    scale = relation_pri / np.sqrt(DK)
    abd = jnp.zeros((R, OUT, OUT), f32)
    mbd = jnp.zeros((R, OUT, OUT), f32)
    for h in range(H):
        sl = slice(h * DK, (h + 1) * DK)
        abd = abd.at[:, sl, sl].set(relation_att[:, h] * scale[:, h, None, None])
        mbd = mbd.at[:, sl, sl].set(relation_msg[:, h])

    gkvc, gqc = _make_gather(P, OUT, 2 * H)(ssrc, sdst_t, pos_t, kvc_t, qc_t)

    selm = np.zeros((OUT, H), np.float32)
    for h in range(H):
        selm[h * DK:(h + 1) * DK, h] = 1.0
    selm = jnp.asarray(selm)
    beta = jax.nn.sigmoid(weight[0]).reshape(1)

    grid4 = (NB,)
    gspec = pltpu.PrefetchScalarGridSpec(
        num_scalar_prefetch=1,
        grid=grid4,
        in_specs=[
            bspec((BE, 3 * OUT), lambda i, rel: (i, 0)),
            bspec((BE, 2 * OUT), lambda i, rel: (i, 0)),
            bspec((1, OUT, OUT), lambda i, rel: (rel[i], 0, 0)),
            bspec((1, OUT, OUT), lambda i, rel: (rel[i], 0, 0)),
            full(selm),
            pl.BlockSpec(memory_space=pltpu.SMEM),
        ],
        out_specs=[
            bspec((H, BE), lambda i, rel: (0, i)),
            bspec((BE, 16), lambda i, rel: (i, 0)),
            bspec((BE, OUT), lambda i, rel: (i, 0)),
            bspec((1, 1, BE), lambda i, rel: (i, 0, 0)),
            bspec((1, 1, BE), lambda i, rel: (i, 0, 0)),
        ],
    )
    lgT, lg16, mvg, seg3, dst3 = pl.pallas_call(
        functools.partial(_edge_body, R),
        grid_spec=gspec,
        out_shape=[
            jax.ShapeDtypeStruct((H, P), f32),
            jax.ShapeDtypeStruct((P, 16), f32),
            jax.ShapeDtypeStruct((P, OUT), f32),
            jax.ShapeDtypeStruct((NB, 1, BE), jnp.int32),
            jax.ShapeDtypeStruct((NB, 1, BE), jnp.int32),
        ],
    )(rel_blocks, gkvc, gqc, abd, mbd, selm, beta)

    seg = seg3.reshape(P)
    sdst = dst3.reshape(P)
    SEGR = Np * R
    den_h = _make_den(P, Np, R, SEGR, H)(lgT, seg)
    denp16 = jnp.concatenate(
        [jnp.transpose(den_h)[seg], jnp.zeros((P, 8), f32)], axis=1)
    tpart = _make_scatter(P, Np, OUT, H)(lg16, sdst, mvg, denp16)
    den0 = den_h[0].reshape(Np, R)

    out = pl.pallas_call(
        functools.partial(_final_body, T),
        grid=grid1,
        in_specs=[
            bspec((BN, OUT), lambda i: (i, 0)),
            bspec((2, BN, OUT), lambda i: (0, i, 0)),
            bspec((BN, R), lambda i: (i, 0)),
            bspec((BN, OUT), lambda i: (i, 0)),
            full(a_w), full(a_b),
            pl.BlockSpec(memory_space=pltpu.SMEM),
        ],
        out_specs=bspec((BN, OUT), lambda i: (i, 0)),
        out_shape=jax.ShapeDtypeStruct((Np, OUT), f32),
    )(ntb, tpart, den0, xp, a_w, a_b, skip)
    return out[:N]


# speedup vs baseline: 1.0682x; 1.0682x over previous
import functools

import jax
import jax.numpy as jnp
import numpy as np
from jax import lax
from jax.experimental import pallas as pl
from jax.experimental.pallas import tpu as pltpu
from jax.experimental.pallas import tpu_sc as plsc

BN = 256
BE = 256
CH = 96


def _proj_body(T, nt_ref, x_ref, kw_ref, kb_ref, qw_ref, qb_ref, vw_ref, vb_ref,
               wc_ref, nta_ref, nta1_ref, kvc_ref, qc_ref):
    xb = x_ref[...]
    ntb = nt_ref[...]
    outs = []
    for (w_ref, b_ref) in ((kw_ref, kb_ref), (qw_ref, qb_ref), (vw_ref, vb_ref)):
        acc = jnp.zeros((xb.shape[0], w_ref.shape[2]), jnp.float32)
        for t in range(T):
            m = jnp.dot(xb, w_ref[t], preferred_element_type=jnp.float32)
            m = m + b_ref[t][None, :]
            acc = jnp.where(ntb == t, m, acc)
        outs.append(acc)
    kacc, qacc, vacc = outs
    wc = wc_ref[...]
    c0 = jnp.dot(kacc, wc[:, 0:8], preferred_element_type=jnp.float32)
    c1 = jnp.dot(qacc, wc[:, 8:16], preferred_element_type=jnp.float32)
    nth = nt_ref[..., 0:8]
    s0 = jnp.zeros_like(c0)
    s1 = jnp.zeros_like(c1)
    for t in range(T):
        s0 = jnp.where(nth == t, nta_ref[t], s0)
        s1 = jnp.where(nth == t, nta1_ref[t], s1)
    c01p = jnp.concatenate(
        [c0 * s0, c1 * s1, jnp.zeros((xb.shape[0], 112), jnp.float32)], axis=1)
    kvc_ref[..., 0:128] = kacc
    kvc_ref[..., 128:256] = vacc
    kvc_ref[..., 256:384] = c01p
    pid = pl.program_id(0)
    row = lax.broadcasted_iota(jnp.int32, c01p.shape, 0).astype(jnp.float32)
    lane = lax.broadcasted_iota(jnp.int32, c01p.shape, 1)
    basef = lax.convert_element_type(pid * c01p.shape[0], jnp.float32)
    ids = jnp.where(lane == 16, row + basef, 0.0)
    qc_ref[..., 0:128] = qacc
    qc_ref[..., 128:256] = c01p + ids


def _edge_body(R, rel_ref, kvc_ref, qc_ref, a_ref, m_ref,
               sel_ref, beta_ref, lg_ref, le_ref, mv_ref, seg_ref, dst_ref):
    i = pl.program_id(0)
    r = rel_ref[i]
    kg = kvc_ref[..., 0:128]
    vg = kvc_ref[..., 128:256]
    qg = qc_ref[..., 0:128]
    z = jnp.dot(kg, a_ref[0], preferred_element_type=jnp.float32)
    att = jnp.dot(z * qg, sel_ref[...], preferred_element_type=jnp.float32)
    att2 = kvc_ref[..., 256:264] + qc_ref[..., 136:144]
    att2 = jnp.where(att2 >= 0, att2, 0.01 * att2)
    lg = att + beta_ref[0] * att2
    lg_ref[...] = jnp.transpose(lg)
    le_ref[...] = jnp.concatenate(
        [lg, jnp.zeros((lg.shape[0], 8), jnp.float32)], axis=1)
    mv_ref[...] = jnp.dot(vg, m_ref[0], preferred_element_type=jnp.float32)
    dt = jnp.transpose(qc_ref[..., 144:145]).astype(jnp.int32)
    dst_ref[0] = dt
    seg_ref[0] = dt * R + r


def _final_body(T, nt_ref, t_ref, d0_ref, x_ref, aw_ref, ab_ref, skip_ref, o_ref):
    pres = (d0_ref[...] > 0).astype(jnp.float32)
    dn = jnp.maximum(jnp.sum(pres, axis=1, keepdims=True), 1.0)
    tb = (t_ref[0] + t_ref[1]) / dn
    ntb = nt_ref[...]
    acc = jnp.zeros_like(tb)
    al = jnp.zeros_like(tb)
    for t in range(T):
        m = jnp.dot(tb, aw_ref[t], preferred_element_type=jnp.float32) + ab_ref[t][None, :]
        acc = jnp.where(ntb == t, m, acc)
        al = jnp.where(ntb == t, jax.nn.sigmoid(skip_ref[t]), al)
    o_ref[...] = acc * al + x_ref[...] * (1.0 - al)


def _cnt_body(nt_ref, o_ref, acc_ref):
    i = pl.program_id(0)

    @pl.when(i == 0)
    def _():
        acc_ref[...] = jnp.zeros_like(acc_ref)

    etb = jnp.broadcast_to(nt_ref[0].astype(jnp.float32), (40, 512))
    rid = lax.broadcasted_iota(jnp.int32, (40, 512), 0).astype(jnp.float32)
    oh = (etb == rid).astype(jnp.float32)
    acc_ref[...] = acc_ref[...] + jnp.broadcast_to(
        jnp.sum(oh, axis=1, keepdims=True), (40, 128))
    o_ref[...] = acc_ref[...]


def _rank_body(et_ref, po_ref, lt_ref, pp_ref, acc_ref):
    i = pl.program_id(0)

    @pl.when(i == 0)
    def _():
        acc_ref[...] = jnp.zeros_like(acc_ref)

    etb = jnp.broadcast_to(et_ref[0].astype(jnp.float32), (40, 512))
    rid = lax.broadcasted_iota(jnp.int32, (40, 512), 0).astype(jnp.float32)
    oh = (etb == rid).astype(jnp.float32)
    cum = jnp.dot(oh.astype(jnp.bfloat16), lt_ref[...],
                  preferred_element_type=jnp.float32)
    rank = jnp.sum(oh * cum, axis=0, keepdims=True)
    base = jnp.sum(oh * acc_ref[...], axis=0, keepdims=True)
    posel = jnp.sum(oh * po_ref[...], axis=0, keepdims=True)
    pp_ref[0] = (rank + base + posel).astype(jnp.int32)
    acc_ref[...] = acc_ref[...] + jnp.broadcast_to(
        jnp.sum(oh, axis=1, keepdims=True), (40, 512))


def _make_gather(P, OUT, CW):
    NW = 32
    PT = P // NW
    NIT = PT // CH
    mesh = plsc.VectorSubcoreMesh(core_axis_name="c", subcore_axis_name="s")
    f32 = jnp.float32

    @functools.partial(
        pl.kernel, mesh=mesh,
        out_type=[
            jax.ShapeDtypeStruct((P, 3 * OUT), f32),
            jax.ShapeDtypeStruct((P, 2 * OUT), f32),
        ],
        scratch_types=[
            pltpu.VMEM((CH,), jnp.int32),
            pltpu.VMEM((CH,), jnp.int32),
            pltpu.VMEM((CH,), jnp.int32),
            pltpu.VMEM((CH,), jnp.int32),
            pltpu.VMEM((CH,), jnp.int32),
            pltpu.VMEM((CH,), jnp.int32),
            pltpu.VMEM((CH, 3 * OUT), f32),
            pltpu.VMEM((CH, 3 * OUT), f32),
            pltpu.VMEM((CH, 2 * OUT), f32),
            pltpu.VMEM((CH, 2 * OUT), f32),
            pltpu.SemaphoreType.DMA,
            pltpu.SemaphoreType.DMA,
            pltpu.SemaphoreType.DMA,
            pltpu.SemaphoreType.DMA,
        ],
    )
    def g(ssrc_h, sdst_h, pos_h, kvct_h, qct_h, kvc_h, qc_h,
          isrc, isrc2, idst, idst2, ipos, ipos2, rbuf, rbuf2, qbuf, qbuf2,
          sem, sem2, sem3, sem4):
        c = lax.axis_index("c")
        s = lax.axis_index("s")
        base = (s * 2 + c) * PT

        def one(off, si, di, pi, rb, qb, sg1, sg2):
            pltpu.sync_copy(ssrc_h.at[pl.ds(off, CH)], si)
            pltpu.sync_copy(sdst_h.at[pl.ds(off, CH)], di)
            pltpu.sync_copy(pos_h.at[pl.ds(off, CH)], pi)
            gk = pltpu.async_copy(kvct_h.at[si], rb, sg1)
            gq = pltpu.async_copy(qct_h.at[di], qb, sg2)
            return gk, gq

        def scat(pi, rb, qb, sg1, sg2):
            sk = pltpu.async_copy(rb, kvc_h.at[pi], sg1)
            sq = pltpu.async_copy(qb, qc_h.at[pi], sg2)
            return sk, sq

        def pair(p, carry):
            offa = base + (2 * p) * CH
            offb = offa + CH
            ga = one(offa, isrc, idst, ipos, rbuf, qbuf, sem, sem2)
            gb = one(offb, isrc2, idst2, ipos2, rbuf2, qbuf2, sem3, sem4)
            ga[0].wait()
            ga[1].wait()
            sa = scat(ipos, rbuf, qbuf, sem, sem2)
            gb[0].wait()
            gb[1].wait()
            sb = scat(ipos2, rbuf2, qbuf2, sem3, sem4)
            sa[0].wait()
            sa[1].wait()
            sb[0].wait()
            sb[1].wait()
            return carry

        lax.fori_loop(0, NIT // 2, pair, 0)
        if NIT % 2:
            off = base + (NIT - 1) * CH
            gk, gq = one(off, isrc, idst, ipos, rbuf, qbuf, sem, sem2)
            gk.wait()
            gq.wait()
            sk, sq = scat(ipos, rbuf, qbuf, sem, sem2)
            sk.wait()
            sq.wait()

    return g


def _make_den(P, Np, R, SEGR, H):
    CH2 = 768
    NIT = P // CH2
    QR = SEGR // 4
    TBL = QR + 16
    mesh = plsc.VectorSubcoreMesh(core_axis_name="c", subcore_axis_name="s")
    f32 = jnp.float32

    @functools.partial(
        pl.kernel, mesh=mesh,
        compiler_params=pltpu.CompilerParams(needs_layout_passes=False),
        out_type=jax.ShapeDtypeStruct((H, SEGR), f32),
        scratch_types=[
            pltpu.VMEM((TBL,), f32),
            pltpu.VMEM((CH2,), jnp.int32),
            pltpu.VMEM((CH2,), f32),
            pltpu.VMEM((CH2,), jnp.int32),
            pltpu.VMEM((CH2,), f32),
            pltpu.SemaphoreType.DMA,
            pltpu.SemaphoreType.DMA,
        ],
    )
    def g(lg_h, seg_h, den_h, tbl, segb, lgb, segb2, lgb2, sem, sem2):
        c = lax.axis_index("c")
        s = lax.axis_index("s")
        combo = c * 16 + s
        q = combo // H
        h = combo % H
        qbase = q * QR

        def zrow(i, carry):
            tbl[pl.ds(i * 16, 16)] = jnp.zeros((16,), f32)
            return carry
        lax.fori_loop(0, TBL // 16, zrow, 0)

        def load(off, sb, lb, sg):
            c1 = pltpu.async_copy(seg_h.at[pl.ds(off, CH2)], sb, sg)
            c2 = pltpu.async_copy(lg_h.at[h, pl.ds(off, CH2)], lb, sg)
            return c1, c2

        def compute(sb, lb):
            for v in range(CH2 // 16):
                sv = sb[pl.ds(v * 16, 16)]
                lv = sv - qbase
                ok = (lv >= 0) & (lv < QR)
                li = jnp.where(ok, lv, QR)
                ex = jnp.exp(lb[pl.ds(v * 16, 16)])
                plsc.addupdate_scatter(tbl, [li], ex)

        def pair(p, carry):
            offa = (2 * p) * CH2
            la = load(offa, segb, lgb, sem)
            lb_ = load(offa + CH2, segb2, lgb2, sem2)
            la[0].wait()
            la[1].wait()
            compute(segb, lgb)
            lb_[0].wait()
            lb_[1].wait()
            compute(segb2, lgb2)
            return carry
        lax.fori_loop(0, NIT // 2, pair, 0)

        pltpu.sync_copy(tbl.at[pl.ds(0, QR)], den_h.at[h, pl.ds(qbase, QR)])

    return g


def _make_scatter(P, Np, OUT, H):
    P2 = P // 2
    PT = P2 // 16
    NIT = PT // CH
    TR = Np // 16
    mesh = plsc.VectorSubcoreMesh(core_axis_name="c", subcore_axis_name="s")
    f32 = jnp.float32

    @functools.partial(
        pl.kernel, mesh=mesh,
        out_type=jax.ShapeDtypeStruct((2, Np, OUT), f32),
        compiler_params=pltpu.CompilerParams(needs_layout_passes=False),
        scratch_types=[
            pltpu.VMEM_SHARED((Np, OUT), f32),
            pltpu.VMEM((32, OUT), f32),
            pltpu.VMEM((CH,), jnp.int32),
            pltpu.VMEM((CH, 16), f32),
            pltpu.VMEM((CH, 16), f32),
            pltpu.VMEM((CH, OUT), f32),
            pltpu.SemaphoreType.DMA,
        ],
    )
    def g(lg_h, dst_h, mv_h, denp_h, tp_h, table, zbuf, dstb, lgb,
          denb, mvb, sem):
        c = lax.axis_index("c")
        s = lax.axis_index("s")

        def zrow(i, carry):
            for j in range(OUT // 16):
                zbuf[i, pl.ds(j * 16, 16)] = jnp.zeros((16,), f32)
            return carry
        lax.fori_loop(0, 32, zrow, 0)

        def zit(j, carry):
            pltpu.sync_copy(zbuf, table.at[pl.ds(s * TR + j * 32, 32)])
            return carry
        lax.fori_loop(0, TR // 32, zit, 0)
        plsc.subcore_barrier()

        ebase = c * P2 + s * PT

        def eit(j, carry):
            off = ebase + j * CH
            pltpu.sync_copy(dst_h.at[pl.ds(off, CH)], dstb)
            pltpu.sync_copy(lg_h.at[pl.ds(off, CH)], lgb)
            pltpu.sync_copy(denp_h.at[pl.ds(off, CH)], denb)
            pltpu.sync_copy(mv_h.at[pl.ds(off, CH)], mvb)

            def erow(i, carry2):
                a = jnp.exp(lgb[i]) / jnp.maximum(denb[i], 1e-9)
                for h in range(H):
                    mvb[i, pl.ds(h * 16, 16)] = mvb[i, pl.ds(h * 16, 16)] * a[h]
                return carry2
            lax.fori_loop(0, CH, erow, 0, unroll=2)
            pltpu.sync_copy(mvb, table.at[dstb], add=True)
            return carry
        lax.fori_loop(0, NIT, eit, 0)
        plsc.subcore_barrier()

        pltpu.sync_copy(table.at[pl.ds(s * TR, TR)],
                        tp_h.at[c, pl.ds(s * TR, TR)])

    return g


def kernel(x, edge_index, edge_type, node_type, k_w, k_b, q_w, q_b, v_w, v_b,
           a_w, a_b, relation_pri, relation_att, relation_msg, node_type_att,
           node_type_att1, skip, weight, attn_fc_w):
    N, IN = x.shape
    T, _, OUT = k_w.shape
    R, H, DK, _ = relation_att.shape
    E = edge_index.shape[1]
    Np = ((N + BN - 1) // BN) * BN
    NBn = Np // BN
    P = E + R * BE
    NB = P // BE

    f32 = jnp.float32
    xp = jnp.pad(x, ((0, Np - N), (0, 0)))
    ntp = jnp.pad(node_type, (0, Np - N)).astype(jnp.int32)
    ntb = jnp.broadcast_to(ntp.astype(f32)[:, None], (Np, OUT))

    w1 = attn_fc_w[:DK]
    w2 = attn_fc_w[DK:]
    eyeh = np.zeros((OUT, 2 * H), np.float32)
    for h in range(H):
        eyeh[h * DK:(h + 1) * DK, h] = 1.0
        eyeh[h * DK:(h + 1) * DK, H + h] = 1.0
    eyeh = jnp.asarray(eyeh)
    wc = jnp.concatenate([eyeh[:, :H] * jnp.tile(w2, H)[:, None],
                          eyeh[:, H:] * jnp.tile(w1, H)[:, None]], axis=1)

    bspec = lambda bs, im: pl.BlockSpec(bs, im)
    full = lambda arr: pl.BlockSpec(arr.shape, lambda *a: tuple(0 for _ in arr.shape))
    grid1 = (NBn,)
    kvc_t, qc_t = pl.pallas_call(
        functools.partial(_proj_body, T),
        grid=grid1,
        in_specs=[
            bspec((BN, OUT), lambda i: (i, 0)),
            bspec((BN, IN), lambda i: (i, 0)),
            full(k_w), full(k_b), full(q_w), full(q_b), full(v_w), full(v_b),
            full(wc),
            pl.BlockSpec(memory_space=pltpu.SMEM),
            pl.BlockSpec(memory_space=pltpu.SMEM),
        ],
        out_specs=[
            bspec((BN, 3 * OUT), lambda i: (i, 0)),
            bspec((BN, 2 * OUT), lambda i: (i, 0)),
        ],
        out_shape=[
            jax.ShapeDtypeStruct((Np, 3 * OUT), f32),
            jax.ShapeDtypeStruct((Np, 2 * OUT), f32),
        ],
    )(ntb, xp, k_w, k_b, q_w, q_b, v_w, v_b, wc, node_type_att, node_type_att1)

    src = edge_index[0]
    dst = edge_index[1]
    EBK = E // 512
    et3 = edge_type.astype(jnp.int32).reshape(EBK, 1, 512)
    cnt40 = pl.pallas_call(
        _cnt_body,
        grid=(EBK,),
        in_specs=[bspec((1, 1, 512), lambda i: (i, 0, 0))],
        out_specs=bspec((40, 128), lambda i: (0, 0)),
        out_shape=jax.ShapeDtypeStruct((40, 128), f32),
        scratch_shapes=[pltpu.VMEM((40, 128), f32)],
    )(et3)
    cnt = cnt40[:R, 0].astype(jnp.int32)
    cap = ((cnt + BE - 1) // BE) * BE
    scap = jnp.cumsum(cap)
    po = jnp.concatenate([jnp.zeros((1,), jnp.int32), scap])[:R]
    total_used = scap[R - 1]
    po40 = jnp.broadcast_to(
        jnp.pad(po.astype(f32), (0, 40 - R))[:, None], (40, 512))
    lt = jnp.asarray(np.triu(np.ones((512, 512), np.float32), 1).T,
                     dtype=jnp.bfloat16)
    ppos3 = pl.pallas_call(
        _rank_body,
        grid=(EBK,),
        in_specs=[
            bspec((1, 1, 512), lambda i: (i, 0, 0)),
            bspec((40, 512), lambda i: (0, 0)),
            bspec((512, 512), lambda i: (0, 0)),
        ],
        out_specs=bspec((1, 1, 512), lambda i: (i, 0, 0)),
        out_shape=jax.ShapeDtypeStruct((EBK, 1, 512), jnp.int32),
        scratch_shapes=[pltpu.VMEM((40, 512), f32)],
    )(et3, po40, lt)
    ppos = ppos3.reshape(E)
    jpad = jnp.arange(R * BE, dtype=jnp.int32)
    rr = jpad // BE
    jj = jpad % BE
    navail = cap - cnt
    validp = jj < navail[rr]
    extra_rank = jnp.cumsum((~validp).astype(jnp.int32)) - 1
    pad_pos = jnp.where(validp, po[rr] + cnt[rr] + jj, total_used + extra_rank)
    ssrc = jnp.concatenate([src, jnp.zeros((R * BE,), jnp.int32)])
    sdst_t = jnp.concatenate([dst, jnp.full((R * BE,), N, jnp.int32)])
    pos_t = jnp.concatenate([ppos, pad_pos.astype(jnp.int32)])
    rel_blocks = jnp.clip(
        jnp.searchsorted(scap, jnp.arange(NB, dtype=jnp.int32) * BE,
                         side="right"), 0, R - 1).astype(jnp.int32)

    scale = relation_pri / np.sqrt(DK)
    abd = jnp.zeros((R, OUT, OUT), f32)
    mbd = jnp.zeros((R, OUT, OUT), f32)
    for h in range(H):
        sl = slice(h * DK, (h + 1) * DK)
        abd = abd.at[:, sl, sl].set(relation_att[:, h] * scale[:, h, None, None])
        mbd = mbd.at[:, sl, sl].set(relation_msg[:, h])

    gkvc, gqc = _make_gather(P, OUT, 2 * H)(ssrc, sdst_t, pos_t, kvc_t, qc_t)

    selm = np.zeros((OUT, H), np.float32)
    for h in range(H):
        selm[h * DK:(h + 1) * DK, h] = 1.0
    selm = jnp.asarray(selm)
    beta = jax.nn.sigmoid(weight[0]).reshape(1)

    grid4 = (NB,)
    gspec = pltpu.PrefetchScalarGridSpec(
        num_scalar_prefetch=1,
        grid=grid4,
        in_specs=[
            bspec((BE, 3 * OUT), lambda i, rel: (i, 0)),
            bspec((BE, 2 * OUT), lambda i, rel: (i, 0)),
            bspec((1, OUT, OUT), lambda i, rel: (rel[i], 0, 0)),
            bspec((1, OUT, OUT), lambda i, rel: (rel[i], 0, 0)),
            full(selm),
            pl.BlockSpec(memory_space=pltpu.SMEM),
        ],
        out_specs=[
            bspec((H, BE), lambda i, rel: (0, i)),
            bspec((BE, 16), lambda i, rel: (i, 0)),
            bspec((BE, OUT), lambda i, rel: (i, 0)),
            bspec((1, 1, BE), lambda i, rel: (i, 0, 0)),
            bspec((1, 1, BE), lambda i, rel: (i, 0, 0)),
        ],
    )
    lgT, lg16, mvg, seg3, dst3 = pl.pallas_call(
        functools.partial(_edge_body, R),
        grid_spec=gspec,
        out_shape=[
            jax.ShapeDtypeStruct((H, P), f32),
            jax.ShapeDtypeStruct((P, 16), f32),
            jax.ShapeDtypeStruct((P, OUT), f32),
            jax.ShapeDtypeStruct((NB, 1, BE), jnp.int32),
            jax.ShapeDtypeStruct((NB, 1, BE), jnp.int32),
        ],
    )(rel_blocks, gkvc, gqc, abd, mbd, selm, beta)

    seg = seg3.reshape(P)
    sdst = dst3.reshape(P)
    SEGR = Np * R
    den_h = _make_den(P, Np, R, SEGR, H)(lgT, seg)
    denp16 = jnp.concatenate(
        [jnp.transpose(den_h)[seg], jnp.zeros((P, 8), f32)], axis=1)
    tpart = _make_scatter(P, Np, OUT, H)(lg16, sdst, mvg, denp16)
    den0 = den_h[0].reshape(Np, R)

    out = pl.pallas_call(
        functools.partial(_final_body, T),
        grid=grid1,
        in_specs=[
            bspec((BN, OUT), lambda i: (i, 0)),
            bspec((2, BN, OUT), lambda i: (0, i, 0)),
            bspec((BN, R), lambda i: (i, 0)),
            bspec((BN, OUT), lambda i: (i, 0)),
            full(a_w), full(a_b),
            pl.BlockSpec(memory_space=pltpu.SMEM),
        ],
        out_specs=bspec((BN, OUT), lambda i: (i, 0)),
        out_shape=jax.ShapeDtypeStruct((Np, OUT), f32),
    )(ntb, tpart, den0, xp, a_w, a_b, skip)
    return out[:N]


# speedup vs baseline: 1.1098x; 1.0389x over previous
import functools

import jax
import jax.numpy as jnp
import numpy as np
from jax import lax
from jax.experimental import pallas as pl
from jax.experimental.pallas import tpu as pltpu
from jax.experimental.pallas import tpu_sc as plsc

BN = 256
BE = 256
CH = 96


def _proj_body(T, nt_ref, x_ref, kw_ref, kb_ref, qw_ref, qb_ref, vw_ref, vb_ref,
               wc_ref, nta_ref, nta1_ref, kvc_ref, qc_ref):
    xb = x_ref[...]
    ntb = nt_ref[...]
    outs = []
    for (w_ref, b_ref) in ((kw_ref, kb_ref), (qw_ref, qb_ref), (vw_ref, vb_ref)):
        acc = jnp.zeros((xb.shape[0], w_ref.shape[2]), jnp.float32)
        for t in range(T):
            m = jnp.dot(xb, w_ref[t], preferred_element_type=jnp.float32)
            m = m + b_ref[t][None, :]
            acc = jnp.where(ntb == t, m, acc)
        outs.append(acc)
    kacc, qacc, vacc = outs
    wc = wc_ref[...]
    c0 = jnp.dot(kacc, wc[:, 0:8], preferred_element_type=jnp.float32)
    c1 = jnp.dot(qacc, wc[:, 8:16], preferred_element_type=jnp.float32)
    nth = nt_ref[..., 0:8]
    s0 = jnp.zeros_like(c0)
    s1 = jnp.zeros_like(c1)
    for t in range(T):
        s0 = jnp.where(nth == t, nta_ref[t], s0)
        s1 = jnp.where(nth == t, nta1_ref[t], s1)
    c01p = jnp.concatenate(
        [c0 * s0, c1 * s1, jnp.zeros((xb.shape[0], 112), jnp.float32)], axis=1)
    kvc_ref[..., 0:128] = kacc
    kvc_ref[..., 128:256] = vacc
    kvc_ref[..., 256:384] = c01p
    pid = pl.program_id(0)
    row = lax.broadcasted_iota(jnp.int32, c01p.shape, 0).astype(jnp.float32)
    lane = lax.broadcasted_iota(jnp.int32, c01p.shape, 1)
    basef = lax.convert_element_type(pid * c01p.shape[0], jnp.float32)
    ids = jnp.where(lane == 16, row + basef, 0.0)
    qc_ref[..., 0:128] = qacc
    qc_ref[..., 128:256] = c01p + ids


def _edge_body(R, rel_ref, kvc_ref, qc_ref, a_ref, m_ref,
               sel_ref, beta_ref, lg_ref, le_ref, mv_ref, seg_ref, dst_ref):
    i = pl.program_id(0)
    r = rel_ref[i]
    kg = kvc_ref[..., 0:128]
    vg = kvc_ref[..., 128:256]
    qg = qc_ref[..., 0:128]
    z = jnp.dot(kg, a_ref[0], preferred_element_type=jnp.float32)
    att = jnp.dot(z * qg, sel_ref[...], preferred_element_type=jnp.float32)
    att2 = kvc_ref[..., 256:264] + qc_ref[..., 136:144]
    att2 = jnp.where(att2 >= 0, att2, 0.01 * att2)
    lg = att + beta_ref[0] * att2
    lg_ref[...] = jnp.transpose(lg)
    le_ref[...] = jnp.concatenate(
        [lg, jnp.zeros((lg.shape[0], 8), jnp.float32)], axis=1)
    mv_ref[...] = jnp.dot(vg, m_ref[0], preferred_element_type=jnp.float32)
    dt = jnp.transpose(qc_ref[..., 144:145]).astype(jnp.int32)
    dst_ref[0] = dt
    seg_ref[0] = dt * R + r


def _final_body(T, nt_ref, t_ref, d0_ref, x_ref, aw_ref, ab_ref, skip_ref, o_ref):
    pres = (d0_ref[...] > 0).astype(jnp.float32)
    dn = jnp.maximum(jnp.sum(pres, axis=1, keepdims=True), 1.0)
    tb = (t_ref[0] + t_ref[1]) / dn
    ntb = nt_ref[...]
    acc = jnp.zeros_like(tb)
    al = jnp.zeros_like(tb)
    for t in range(T):
        m = jnp.dot(tb, aw_ref[t], preferred_element_type=jnp.float32) + ab_ref[t][None, :]
        acc = jnp.where(ntb == t, m, acc)
        al = jnp.where(ntb == t, jax.nn.sigmoid(skip_ref[t]), al)
    o_ref[...] = acc * al + x_ref[...] * (1.0 - al)


def _cnt_body(nt_ref, o_ref, acc_ref):
    i = pl.program_id(0)

    @pl.when(i == 0)
    def _():
        acc_ref[...] = jnp.zeros_like(acc_ref)

    etb = jnp.broadcast_to(nt_ref[0].astype(jnp.float32), (40, 512))
    rid = lax.broadcasted_iota(jnp.int32, (40, 512), 0).astype(jnp.float32)
    oh = (etb == rid).astype(jnp.float32)
    acc_ref[...] = acc_ref[...] + jnp.broadcast_to(
        jnp.sum(oh, axis=1, keepdims=True), (40, 128))
    o_ref[...] = acc_ref[...]


def _rank_body(et_ref, po_ref, lt_ref, pp_ref, acc_ref):
    i = pl.program_id(0)

    @pl.when(i == 0)
    def _():
        acc_ref[...] = jnp.zeros_like(acc_ref)

    etb = jnp.broadcast_to(et_ref[0].astype(jnp.float32), (40, 512))
    rid = lax.broadcasted_iota(jnp.int32, (40, 512), 0).astype(jnp.float32)
    oh = (etb == rid).astype(jnp.float32)
    cum = jnp.dot(oh.astype(jnp.bfloat16), lt_ref[...],
                  preferred_element_type=jnp.float32)
    rank = jnp.sum(oh * cum, axis=0, keepdims=True)
    base = jnp.sum(oh * acc_ref[...], axis=0, keepdims=True)
    posel = jnp.sum(oh * po_ref[...], axis=0, keepdims=True)
    pp_ref[0] = (rank + base + posel).astype(jnp.int32)
    acc_ref[...] = acc_ref[...] + jnp.broadcast_to(
        jnp.sum(oh, axis=1, keepdims=True), (40, 512))


def _make_gather(P, OUT, CW):
    NW = 32
    PT = P // NW
    NIT = PT // CH
    mesh = plsc.VectorSubcoreMesh(core_axis_name="c", subcore_axis_name="s")
    f32 = jnp.float32

    @functools.partial(
        pl.kernel, mesh=mesh,
        out_type=[
            jax.ShapeDtypeStruct((P, 3 * OUT), f32),
            jax.ShapeDtypeStruct((P, 2 * OUT), f32),
        ],
        scratch_types=[
            pltpu.VMEM((CH,), jnp.int32),
            pltpu.VMEM((CH,), jnp.int32),
            pltpu.VMEM((CH,), jnp.int32),
            pltpu.VMEM((CH,), jnp.int32),
            pltpu.VMEM((CH,), jnp.int32),
            pltpu.VMEM((CH,), jnp.int32),
            pltpu.VMEM((CH, 3 * OUT), f32),
            pltpu.VMEM((CH, 3 * OUT), f32),
            pltpu.VMEM((CH, 2 * OUT), f32),
            pltpu.VMEM((CH, 2 * OUT), f32),
            pltpu.SemaphoreType.DMA,
            pltpu.SemaphoreType.DMA,
            pltpu.SemaphoreType.DMA,
            pltpu.SemaphoreType.DMA,
        ],
    )
    def g(ssrc_h, sdst_h, pos_h, kvct_h, qct_h, kvc_h, qc_h,
          isrc, isrc2, idst, idst2, ipos, ipos2, rbuf, rbuf2, qbuf, qbuf2,
          sem, sem2, sem3, sem4):
        c = lax.axis_index("c")
        s = lax.axis_index("s")
        base = (s * 2 + c) * PT

        def one(off, si, di, pi, rb, qb, sg1, sg2):
            pltpu.sync_copy(ssrc_h.at[pl.ds(off, CH)], si)
            pltpu.sync_copy(sdst_h.at[pl.ds(off, CH)], di)
            pltpu.sync_copy(pos_h.at[pl.ds(off, CH)], pi)
            gk = pltpu.async_copy(kvct_h.at[si], rb, sg1)
            gq = pltpu.async_copy(qct_h.at[di], qb, sg2)
            return gk, gq

        def scat(pi, rb, qb, sg1, sg2):
            sk = pltpu.async_copy(rb, kvc_h.at[pi], sg1)
            sq = pltpu.async_copy(qb, qc_h.at[pi], sg2)
            return sk, sq

        def pair(p, carry):
            offa = base + (2 * p) * CH
            offb = offa + CH
            ga = one(offa, isrc, idst, ipos, rbuf, qbuf, sem, sem2)
            gb = one(offb, isrc2, idst2, ipos2, rbuf2, qbuf2, sem3, sem4)
            ga[0].wait()
            ga[1].wait()
            sa = scat(ipos, rbuf, qbuf, sem, sem2)
            gb[0].wait()
            gb[1].wait()
            sb = scat(ipos2, rbuf2, qbuf2, sem3, sem4)
            sa[0].wait()
            sa[1].wait()
            sb[0].wait()
            sb[1].wait()
            return carry

        lax.fori_loop(0, NIT // 2, pair, 0)
        if NIT % 2:
            off = base + (NIT - 1) * CH
            gk, gq = one(off, isrc, idst, ipos, rbuf, qbuf, sem, sem2)
            gk.wait()
            gq.wait()
            sk, sq = scat(ipos, rbuf, qbuf, sem, sem2)
            sk.wait()
            sq.wait()

    return g


def _make_den(P, Np, R, SEGR, H):
    CH2 = 768
    NIT = P // CH2
    QR = SEGR // 4
    TBL = QR + 16
    mesh = plsc.VectorSubcoreMesh(core_axis_name="c", subcore_axis_name="s")
    f32 = jnp.float32

    @functools.partial(
        pl.kernel, mesh=mesh,
        compiler_params=pltpu.CompilerParams(needs_layout_passes=False),
        out_type=jax.ShapeDtypeStruct((H, SEGR), f32),
        scratch_types=[
            pltpu.VMEM((TBL,), f32),
            pltpu.VMEM((CH2,), jnp.int32),
            pltpu.VMEM((CH2,), f32),
            pltpu.VMEM((CH2,), jnp.int32),
            pltpu.VMEM((CH2,), f32),
            pltpu.SemaphoreType.DMA,
            pltpu.SemaphoreType.DMA,
        ],
    )
    def g(lg_h, seg_h, den_h, tbl, segb, lgb, segb2, lgb2, sem, sem2):
        c = lax.axis_index("c")
        s = lax.axis_index("s")
        combo = c * 16 + s
        q = combo // H
        h = combo % H
        qbase = q * QR

        def zrow(i, carry):
            tbl[pl.ds(i * 16, 16)] = jnp.zeros((16,), f32)
            return carry
        lax.fori_loop(0, TBL // 16, zrow, 0)

        def load(off, sb, lb, sg):
            c1 = pltpu.async_copy(seg_h.at[pl.ds(off, CH2)], sb, sg)
            c2 = pltpu.async_copy(lg_h.at[h, pl.ds(off, CH2)], lb, sg)
            return c1, c2

        def compute(sb, lb):
            for v in range(CH2 // 16):
                sv = sb[pl.ds(v * 16, 16)]
                lv = sv - qbase
                ok = (lv >= 0) & (lv < QR)
                li = jnp.where(ok, lv, QR)
                ex = jnp.exp(lb[pl.ds(v * 16, 16)])
                plsc.addupdate_scatter(tbl, [li], ex)

        def pair(p, carry):
            offa = (2 * p) * CH2
            la = load(offa, segb, lgb, sem)
            lb_ = load(offa + CH2, segb2, lgb2, sem2)
            la[0].wait()
            la[1].wait()
            compute(segb, lgb)
            lb_[0].wait()
            lb_[1].wait()
            compute(segb2, lgb2)
            return carry
        lax.fori_loop(0, NIT // 2, pair, 0)

        pltpu.sync_copy(tbl.at[pl.ds(0, QR)], den_h.at[h, pl.ds(qbase, QR)])

    return g


def _make_scatter(P, Np, OUT, H):
    P2 = P // 2
    PT = P2 // 16
    CS = 48
    NIT = PT // CS
    TR = Np // 16
    mesh = plsc.VectorSubcoreMesh(core_axis_name="c", subcore_axis_name="s")
    f32 = jnp.float32

    @functools.partial(
        pl.kernel, mesh=mesh,
        out_type=jax.ShapeDtypeStruct((2, Np, OUT), f32),
        compiler_params=pltpu.CompilerParams(needs_layout_passes=False),
        scratch_types=[
            pltpu.VMEM_SHARED((Np, OUT), f32),
            pltpu.VMEM((32, OUT), f32),
            pltpu.VMEM((CS,), jnp.int32),
            pltpu.VMEM((CS, 16), f32),
            pltpu.VMEM((CS, 16), f32),
            pltpu.VMEM((CS, OUT), f32),
            pltpu.VMEM((CS,), jnp.int32),
            pltpu.VMEM((CS, 16), f32),
            pltpu.VMEM((CS, 16), f32),
            pltpu.VMEM((CS, OUT), f32),
            pltpu.SemaphoreType.DMA,
            pltpu.SemaphoreType.DMA,
        ],
    )
    def g(lg_h, dst_h, mv_h, denp_h, tp_h, table, zbuf, dstb, lgb,
          denb, mvb, dstb2, lgb2, denb2, mvb2, sem, sem2):
        c = lax.axis_index("c")
        s = lax.axis_index("s")

        def zrow(i, carry):
            for j in range(OUT // 16):
                zbuf[i, pl.ds(j * 16, 16)] = jnp.zeros((16,), f32)
            return carry
        lax.fori_loop(0, 32, zrow, 0)

        def zit(j, carry):
            pltpu.sync_copy(zbuf, table.at[pl.ds(s * TR + j * 32, 32)])
            return carry
        lax.fori_loop(0, TR // 32, zit, 0)
        plsc.subcore_barrier()

        ebase = c * P2 + s * PT

        def load(off, db, lb, nb, mb, sg):
            c1 = pltpu.async_copy(dst_h.at[pl.ds(off, CS)], db, sg)
            c2 = pltpu.async_copy(lg_h.at[pl.ds(off, CS)], lb, sg)
            c3 = pltpu.async_copy(denp_h.at[pl.ds(off, CS)], nb, sg)
            c4 = pltpu.async_copy(mv_h.at[pl.ds(off, CS)], mb, sg)
            return c1, c2, c3, c4

        def compute(lb, nb, mb):
            def erow(i, carry2):
                a = jnp.exp(lb[i]) / jnp.maximum(nb[i], 1e-9)
                for h in range(H):
                    mb[i, pl.ds(h * 16, 16)] = mb[i, pl.ds(h * 16, 16)] * a[h]
                return carry2
            lax.fori_loop(0, CS, erow, 0, unroll=2)

        def pair(p, carry):
            offa = ebase + (2 * p) * CS
            la = load(offa, dstb, lgb, denb, mvb, sem)
            lb_ = load(offa + CS, dstb2, lgb2, denb2, mvb2, sem2)
            for x in la:
                x.wait()
            compute(lgb, denb, mvb)
            pltpu.sync_copy(mvb, table.at[dstb], add=True)
            for x in lb_:
                x.wait()
            compute(lgb2, denb2, mvb2)
            pltpu.sync_copy(mvb2, table.at[dstb2], add=True)
            return carry
        lax.fori_loop(0, NIT // 2, pair, 0)
        plsc.subcore_barrier()

        pltpu.sync_copy(table.at[pl.ds(s * TR, TR)],
                        tp_h.at[c, pl.ds(s * TR, TR)])

    return g


def kernel(x, edge_index, edge_type, node_type, k_w, k_b, q_w, q_b, v_w, v_b,
           a_w, a_b, relation_pri, relation_att, relation_msg, node_type_att,
           node_type_att1, skip, weight, attn_fc_w):
    N, IN = x.shape
    T, _, OUT = k_w.shape
    R, H, DK, _ = relation_att.shape
    E = edge_index.shape[1]
    Np = ((N + BN - 1) // BN) * BN
    NBn = Np // BN
    P = E + R * BE
    NB = P // BE

    f32 = jnp.float32
    xp = jnp.pad(x, ((0, Np - N), (0, 0)))
    ntp = jnp.pad(node_type, (0, Np - N)).astype(jnp.int32)
    ntb = jnp.broadcast_to(ntp.astype(f32)[:, None], (Np, OUT))

    w1 = attn_fc_w[:DK]
    w2 = attn_fc_w[DK:]
    eyeh = np.zeros((OUT, 2 * H), np.float32)
    for h in range(H):
        eyeh[h * DK:(h + 1) * DK, h] = 1.0
        eyeh[h * DK:(h + 1) * DK, H + h] = 1.0
    eyeh = jnp.asarray(eyeh)
    wc = jnp.concatenate([eyeh[:, :H] * jnp.tile(w2, H)[:, None],
                          eyeh[:, H:] * jnp.tile(w1, H)[:, None]], axis=1)

    bspec = lambda bs, im: pl.BlockSpec(bs, im)
    full = lambda arr: pl.BlockSpec(arr.shape, lambda *a: tuple(0 for _ in arr.shape))
    grid1 = (NBn,)
    kvc_t, qc_t = pl.pallas_call(
        functools.partial(_proj_body, T),
        grid=grid1,
        in_specs=[
            bspec((BN, OUT), lambda i: (i, 0)),
            bspec((BN, IN), lambda i: (i, 0)),
            full(k_w), full(k_b), full(q_w), full(q_b), full(v_w), full(v_b),
            full(wc),
            pl.BlockSpec(memory_space=pltpu.SMEM),
            pl.BlockSpec(memory_space=pltpu.SMEM),
        ],
        out_specs=[
            bspec((BN, 3 * OUT), lambda i: (i, 0)),
            bspec((BN, 2 * OUT), lambda i: (i, 0)),
        ],
        out_shape=[
            jax.ShapeDtypeStruct((Np, 3 * OUT), f32),
            jax.ShapeDtypeStruct((Np, 2 * OUT), f32),
        ],
    )(ntb, xp, k_w, k_b, q_w, q_b, v_w, v_b, wc, node_type_att, node_type_att1)

    src = edge_index[0]
    dst = edge_index[1]
    EBK = E // 512
    et3 = edge_type.astype(jnp.int32).reshape(EBK, 1, 512)
    cnt40 = pl.pallas_call(
        _cnt_body,
        grid=(EBK,),
        in_specs=[bspec((1, 1, 512), lambda i: (i, 0, 0))],
        out_specs=bspec((40, 128), lambda i: (0, 0)),
        out_shape=jax.ShapeDtypeStruct((40, 128), f32),
        scratch_shapes=[pltpu.VMEM((40, 128), f32)],
    )(et3)
    cnt = cnt40[:R, 0].astype(jnp.int32)
    cap = ((cnt + BE - 1) // BE) * BE
    scap = jnp.cumsum(cap)
    po = jnp.concatenate([jnp.zeros((1,), jnp.int32), scap])[:R]
    total_used = scap[R - 1]
    po40 = jnp.broadcast_to(
        jnp.pad(po.astype(f32), (0, 40 - R))[:, None], (40, 512))
    lt = jnp.asarray(np.triu(np.ones((512, 512), np.float32), 1).T,
                     dtype=jnp.bfloat16)
    ppos3 = pl.pallas_call(
        _rank_body,
        grid=(EBK,),
        in_specs=[
            bspec((1, 1, 512), lambda i: (i, 0, 0)),
            bspec((40, 512), lambda i: (0, 0)),
            bspec((512, 512), lambda i: (0, 0)),
        ],
        out_specs=bspec((1, 1, 512), lambda i: (i, 0, 0)),
        out_shape=jax.ShapeDtypeStruct((EBK, 1, 512), jnp.int32),
        scratch_shapes=[pltpu.VMEM((40, 512), f32)],
    )(et3, po40, lt)
    ppos = ppos3.reshape(E)
    jpad = jnp.arange(R * BE, dtype=jnp.int32)
    rr = jpad // BE
    jj = jpad % BE
    navail = cap - cnt
    validp = jj < navail[rr]
    extra_rank = jnp.cumsum((~validp).astype(jnp.int32)) - 1
    pad_pos = jnp.where(validp, po[rr] + cnt[rr] + jj, total_used + extra_rank)
    ssrc = jnp.concatenate([src, jnp.zeros((R * BE,), jnp.int32)])
    sdst_t = jnp.concatenate([dst, jnp.full((R * BE,), N, jnp.int32)])
    pos_t = jnp.concatenate([ppos, pad_pos.astype(jnp.int32)])
    rel_blocks = jnp.clip(
        jnp.searchsorted(scap, jnp.arange(NB, dtype=jnp.int32) * BE,
                         side="right"), 0, R - 1).astype(jnp.int32)

    scale = relation_pri / np.sqrt(DK)
    abd = jnp.zeros((R, OUT, OUT), f32)
    mbd = jnp.zeros((R, OUT, OUT), f32)
    for h in range(H):
        sl = slice(h * DK, (h + 1) * DK)
        abd = abd.at[:, sl, sl].set(relation_att[:, h] * scale[:, h, None, None])
        mbd = mbd.at[:, sl, sl].set(relation_msg[:, h])

    gkvc, gqc = _make_gather(P, OUT, 2 * H)(ssrc, sdst_t, pos_t, kvc_t, qc_t)

    selm = np.zeros((OUT, H), np.float32)
    for h in range(H):
        selm[h * DK:(h + 1) * DK, h] = 1.0
    selm = jnp.asarray(selm)
    beta = jax.nn.sigmoid(weight[0]).reshape(1)

    grid4 = (NB,)
    gspec = pltpu.PrefetchScalarGridSpec(
        num_scalar_prefetch=1,
        grid=grid4,
        in_specs=[
            bspec((BE, 3 * OUT), lambda i, rel: (i, 0)),
            bspec((BE, 2 * OUT), lambda i, rel: (i, 0)),
            bspec((1, OUT, OUT), lambda i, rel: (rel[i], 0, 0)),
            bspec((1, OUT, OUT), lambda i, rel: (rel[i], 0, 0)),
            full(selm),
            pl.BlockSpec(memory_space=pltpu.SMEM),
        ],
        out_specs=[
            bspec((H, BE), lambda i, rel: (0, i)),
            bspec((BE, 16), lambda i, rel: (i, 0)),
            bspec((BE, OUT), lambda i, rel: (i, 0)),
            bspec((1, 1, BE), lambda i, rel: (i, 0, 0)),
            bspec((1, 1, BE), lambda i, rel: (i, 0, 0)),
        ],
    )
    lgT, lg16, mvg, seg3, dst3 = pl.pallas_call(
        functools.partial(_edge_body, R),
        grid_spec=gspec,
        out_shape=[
            jax.ShapeDtypeStruct((H, P), f32),
            jax.ShapeDtypeStruct((P, 16), f32),
            jax.ShapeDtypeStruct((P, OUT), f32),
            jax.ShapeDtypeStruct((NB, 1, BE), jnp.int32),
            jax.ShapeDtypeStruct((NB, 1, BE), jnp.int32),
        ],
    )(rel_blocks, gkvc, gqc, abd, mbd, selm, beta)

    seg = seg3.reshape(P)
    sdst = dst3.reshape(P)
    SEGR = Np * R
    den_h = _make_den(P, Np, R, SEGR, H)(lgT, seg)
    denp16 = jnp.concatenate(
        [jnp.transpose(den_h)[seg], jnp.zeros((P, 8), f32)], axis=1)
    tpart = _make_scatter(P, Np, OUT, H)(lg16, sdst, mvg, denp16)
    den0 = den_h[0].reshape(Np, R)

    out = pl.pallas_call(
        functools.partial(_final_body, T),
        grid=grid1,
        in_specs=[
            bspec((BN, OUT), lambda i: (i, 0)),
            bspec((2, BN, OUT), lambda i: (0, i, 0)),
            bspec((BN, R), lambda i: (i, 0)),
            bspec((BN, OUT), lambda i: (i, 0)),
            full(a_w), full(a_b),
            pl.BlockSpec(memory_space=pltpu.SMEM),
        ],
        out_specs=bspec((BN, OUT), lambda i: (i, 0)),
        out_shape=jax.ShapeDtypeStruct((Np, OUT), f32),
    )(ntb, tpart, den0, xp, a_w, a_b, skip)
    return out[:N]
